# baseline (device time: 260526 ns/iter reference)
import jax
import jax.numpy as jnp
from jax import lax
from jax.experimental import pallas as pl
from jax.experimental.pallas import tpu as pltpu

M = 4096
D = 4096
HALF = M // 2
B = 256
NBH = HALF // B
R = 4
T = 3


def kernel(partial, resid, gamma):
    pb = partial.reshape(M, D)

    def body(pb_ref, rb_ref, g_ref, out_ref,
             ohA, ohB, pbA, pbF, ridA, pbB, ridB, outA, outB,
             sy_send, sy_recv, sx_send, sx_recv,
             s_pbF, s_ridA, s_pbB, s_ridB, s_outA, s_outB):
        x = lax.axis_index("x")
        y = lax.axis_index("y")
        z = lax.axis_index("z")
        ynbr = (x, 1 - y, z)
        xnbr = (1 - x, y, z)
        baseA = x * HALF
        baseB = (1 - x) * HALF

        def ysend(j):
            return pltpu.make_async_remote_copy(
                src_ref=pbA.at[j % T],
                dst_ref=ohA.at[j % R],
                send_sem=sy_send.at[j % T],
                recv_sem=sy_recv.at[j % R],
                device_id=ynbr,
                device_id_type=pl.DeviceIdType.MESH,
            )

        def xfwd(j):
            return pltpu.make_async_remote_copy(
                src_ref=ohA.at[j % R],
                dst_ref=ohB.at[j % R],
                send_sem=sx_send.at[j % T],
                recv_sem=sx_recv.at[j % R],
                device_id=xnbr,
                device_id_type=pl.DeviceIdType.MESH,
            )

        def load(hbm, row, dst, sem, wait=False):
            cp = pltpu.make_async_copy(hbm.at[pl.ds(row, B), :], dst, sem)
            cp.wait() if wait else cp.start()

        def store(src, row, sem, wait=False):
            cp = pltpu.make_async_copy(src, out_ref.at[pl.ds(row, B), :], sem)
            cp.wait() if wait else cp.start()

        def norm(v):
            ms = jnp.mean(v * v, axis=-1, keepdims=True)
            return v * lax.rsqrt(ms + 1e-6) * g_ref[...][None, :]

        bsem = pltpu.get_barrier_semaphore()
        for nbr in (ynbr, xnbr):
            pl.semaphore_signal(
                bsem, inc=1, device_id=nbr,
                device_id_type=pl.DeviceIdType.MESH,
            )
        pl.semaphore_wait(bsem, 2)

        load(pb_ref, baseA, pbF.at[0], s_pbF.at[0])
        load(rb_ref, baseA, ridA.at[0], s_ridA.at[0])
        if NBH > 1:
            load(rb_ref, baseA + B, ridA.at[1], s_ridA.at[1])
        load(pb_ref, baseA, pbF.at[0], s_pbF.at[0], wait=True)
        pbA[0, :, :] = pbF[0, :, :].astype(jnp.bfloat16)

        for k in range(NBH + 2):
            if k < NBH:
                ysend(k).start()

            j = k - 1
            if 0 <= j < NBH:
                ysend(j).wait_recv()
                if 0 <= j - T < NBH:
                    xfwd(j - T).wait_send()
                xfwd(j).start()
                load(rb_ref, baseA + j * B, ridA.at[j % 2], s_ridA.at[j % 2],
                     wait=True)
                if j >= 1:
                    store(outA, baseA + (j - 1) * B, s_outA, wait=True)
                yA = (pbA[j % T, :, :].astype(jnp.float32)
                      + ohA[j % R, :, :].astype(jnp.float32)
                      + ridA[j % 2, :, :])
                outA[...] = norm(yA)
                store(outA, baseA + j * B, s_outA)
                if j + 2 < NBH:
                    load(rb_ref, baseA + (j + 2) * B, ridA.at[j % 2],
                         s_ridA.at[j % 2])

            j2 = k - 2
            if 0 <= j2 < NBH:
                xfwd(j2).wait_recv()
                load(pb_ref, baseB + j2 * B, pbB.at[j2 % 2], s_pbB.at[j2 % 2],
                     wait=True)
                load(rb_ref, baseB + j2 * B, ridB.at[j2 % 2],
                     s_ridB.at[j2 % 2], wait=True)
                if j2 >= 1:
                    store(outB, baseB + (j2 - 1) * B, s_outB, wait=True)
                yB = (pbB[j2 % 2, :, :]
                      + ohB[j2 % R, :, :].astype(jnp.float32)
                      + ridB[j2 % 2, :, :])
                outB[...] = norm(yB)
                store(outB, baseB + j2 * B, s_outB)

            if k < NBH:
                load(pb_ref, baseB + k * B, pbB.at[k % 2], s_pbB.at[k % 2])
                load(rb_ref, baseB + k * B, ridB.at[k % 2], s_ridB.at[k % 2])

            if k + 1 < NBH:
                if 0 <= k + 1 - T < NBH:
                    ysend(k + 1 - T).wait_send()
                load(pb_ref, baseA + (k + 1) * B, pbF.at[0], s_pbF.at[0])
                load(pb_ref, baseA + (k + 1) * B, pbF.at[0], s_pbF.at[0],
                     wait=True)
                pbA[(k + 1) % T, :, :] = pbF[0, :, :].astype(jnp.bfloat16)

        for j in range(max(0, NBH - T), NBH):
            ysend(j).wait_send()
        for j in range(max(0, NBH - T), NBH):
            xfwd(j).wait_send()
        store(outA, baseA + (NBH - 1) * B, s_outA, wait=True)
        store(outB, baseB + (NBH - 1) * B, s_outB, wait=True)

    return pl.pallas_call(
        body,
        in_specs=[
            pl.BlockSpec(memory_space=pl.ANY),
            pl.BlockSpec(memory_space=pl.ANY),
            pl.BlockSpec(memory_space=pltpu.VMEM),
        ],
        out_specs=pl.BlockSpec(memory_space=pl.ANY),
        out_shape=jax.ShapeDtypeStruct((M, D), jnp.float32),
        scratch_shapes=[
            pltpu.VMEM((R, B, D), jnp.bfloat16),
            pltpu.VMEM((R, B, D), jnp.bfloat16),
            pltpu.VMEM((T, B, D), jnp.bfloat16),
            pltpu.VMEM((1, B, D), jnp.float32),
            pltpu.VMEM((2, B, D), jnp.float32),
            pltpu.VMEM((2, B, D), jnp.float32),
            pltpu.VMEM((2, B, D), jnp.float32),
            pltpu.VMEM((B, D), jnp.float32),
            pltpu.VMEM((B, D), jnp.float32),
            pltpu.SemaphoreType.DMA((T,)),
            pltpu.SemaphoreType.DMA((R,)),
            pltpu.SemaphoreType.DMA((T,)),
            pltpu.SemaphoreType.DMA((R,)),
            pltpu.SemaphoreType.DMA((1,)),
            pltpu.SemaphoreType.DMA((2,)),
            pltpu.SemaphoreType.DMA((2,)),
            pltpu.SemaphoreType.DMA((2,)),
            pltpu.SemaphoreType.DMA,
            pltpu.SemaphoreType.DMA,
        ],
        compiler_params=pltpu.CompilerParams(
            collective_id=0, vmem_limit_bytes=63 * 1024 * 1024,
        ),
    )(pb, resid, gamma)


# device time: 259456 ns/iter; 1.0041x vs baseline; 1.0041x over previous
import jax
import jax.numpy as jnp
from jax import lax
from jax.experimental import pallas as pl
from jax.experimental.pallas import tpu as pltpu

M = 4096
D = 4096
HALF = M // 2
B = 256
NBH = HALF // B
R = 4
T = 3


def kernel(partial, resid, gamma):
    pb = partial.reshape(M, D)

    def body(pb_ref, rb_ref, g_ref, out_ref,
             ohA, ohB, pbA, pbF, ridA, pbB, ridB, outS,
             sy_send, sy_recv, sx_send, sx_recv,
             s_pbF, s_ridA, s_pbB, s_ridB, s_out):
        x = lax.axis_index("x")
        y = lax.axis_index("y")
        z = lax.axis_index("z")
        ynbr = (x, 1 - y, z)
        xnbr = (1 - x, y, z)
        baseA = x * HALF
        baseB = (1 - x) * HALF

        def ysend(j):
            return pltpu.make_async_remote_copy(
                src_ref=pbA.at[j % T],
                dst_ref=ohA.at[j % R],
                send_sem=sy_send.at[j % T],
                recv_sem=sy_recv.at[j % R],
                device_id=ynbr,
                device_id_type=pl.DeviceIdType.MESH,
            )

        def xfwd(j):
            return pltpu.make_async_remote_copy(
                src_ref=ohA.at[j % R],
                dst_ref=ohB.at[j % R],
                send_sem=sx_send.at[j % T],
                recv_sem=sx_recv.at[j % R],
                device_id=xnbr,
                device_id_type=pl.DeviceIdType.MESH,
            )

        def load(hbm, row, dst, sem, wait=False):
            cp = pltpu.make_async_copy(hbm.at[pl.ds(row, B), :], dst, sem)
            cp.wait() if wait else cp.start()

        def store(src, row, sem, wait=False):
            cp = pltpu.make_async_copy(src, out_ref.at[pl.ds(row, B), :], sem)
            cp.wait() if wait else cp.start()

        def norm(v):
            ms = jnp.mean(v * v, axis=-1, keepdims=True)
            return v * lax.rsqrt(ms + 1e-6) * g_ref[...][None, :]

        bsem = pltpu.get_barrier_semaphore()
        for nbr in (ynbr, xnbr):
            pl.semaphore_signal(
                bsem, inc=1, device_id=nbr,
                device_id_type=pl.DeviceIdType.MESH,
            )
        pl.semaphore_wait(bsem, 2)

        load(pb_ref, baseA, pbF.at[0], s_pbF.at[0])
        load(rb_ref, baseA, ridA.at[0], s_ridA.at[0])
        if NBH > 1:
            load(rb_ref, baseA + B, ridA.at[1], s_ridA.at[1])
            load(pb_ref, baseA + B, pbF.at[1], s_pbF.at[1])
        load(pb_ref, baseA, pbF.at[0], s_pbF.at[0], wait=True)
        pbA[0, :, :] = pbF[0, :, :].astype(jnp.bfloat16)

        for k in range(NBH + 2):
            if k < NBH:
                ysend(k).start()

            if k + 2 < NBH:
                load(pb_ref, baseA + (k + 2) * B, pbF.at[k % 2],
                     s_pbF.at[k % 2])

            j = k - 1
            if 0 <= j < NBH:
                ysend(j).wait_recv()
                if 0 <= j - T < NBH:
                    xfwd(j - T).wait_send()
                xfwd(j).start()
                load(rb_ref, baseA + j * B, ridA.at[j % 2], s_ridA.at[j % 2],
                     wait=True)
                if j >= 1:
                    store(outS, baseA + (j - 1) * B, s_out, wait=True)
                yA = (pbA[j % T, :, :].astype(jnp.float32)
                      + ohA[j % R, :, :].astype(jnp.float32)
                      + ridA[j % 2, :, :])
                outS[...] = norm(yA)
                store(outS, baseA + j * B, s_out)
                if j + 2 < NBH:
                    load(rb_ref, baseA + (j + 2) * B, ridA.at[j % 2],
                         s_ridA.at[j % 2])

            j2 = k - 2
            if 0 <= j2 < NBH:
                xfwd(j2).wait_recv()
                load(pb_ref, baseB + j2 * B, pbB.at[j2 % 2], s_pbB.at[j2 % 2],
                     wait=True)
                load(rb_ref, baseB + j2 * B, ridB.at[j2 % 2],
                     s_ridB.at[j2 % 2], wait=True)
                store(outS, baseB + j2 * B, s_out, wait=True)
                yB = (pbB[j2 % 2, :, :]
                      + ohB[j2 % R, :, :].astype(jnp.float32)
                      + ridB[j2 % 2, :, :])
                outS[...] = norm(yB)
                store(outS, baseB + j2 * B, s_out)

            if k < NBH:
                load(pb_ref, baseB + k * B, pbB.at[k % 2], s_pbB.at[k % 2])
                load(rb_ref, baseB + k * B, ridB.at[k % 2], s_ridB.at[k % 2])

            if k + 1 < NBH:
                if 0 <= k + 1 - T < NBH:
                    ysend(k + 1 - T).wait_send()
                load(pb_ref, baseA + (k + 1) * B, pbF.at[(k + 1) % 2],
                     s_pbF.at[(k + 1) % 2], wait=True)
                pbA[(k + 1) % T, :, :] = (
                    pbF[(k + 1) % 2, :, :].astype(jnp.bfloat16))

        for j in range(max(0, NBH - T), NBH):
            ysend(j).wait_send()
        for j in range(max(0, NBH - T), NBH):
            xfwd(j).wait_send()
        store(outS, baseB + (NBH - 1) * B, s_out, wait=True)

    return pl.pallas_call(
        body,
        in_specs=[
            pl.BlockSpec(memory_space=pl.ANY),
            pl.BlockSpec(memory_space=pl.ANY),
            pl.BlockSpec(memory_space=pltpu.VMEM),
        ],
        out_specs=pl.BlockSpec(memory_space=pl.ANY),
        out_shape=jax.ShapeDtypeStruct((M, D), jnp.float32),
        scratch_shapes=[
            pltpu.VMEM((R, B, D), jnp.bfloat16),
            pltpu.VMEM((R, B, D), jnp.bfloat16),
            pltpu.VMEM((T, B, D), jnp.bfloat16),
            pltpu.VMEM((2, B, D), jnp.float32),
            pltpu.VMEM((2, B, D), jnp.float32),
            pltpu.VMEM((2, B, D), jnp.float32),
            pltpu.VMEM((2, B, D), jnp.float32),
            pltpu.VMEM((B, D), jnp.float32),
            pltpu.SemaphoreType.DMA((T,)),
            pltpu.SemaphoreType.DMA((R,)),
            pltpu.SemaphoreType.DMA((T,)),
            pltpu.SemaphoreType.DMA((R,)),
            pltpu.SemaphoreType.DMA((2,)),
            pltpu.SemaphoreType.DMA((2,)),
            pltpu.SemaphoreType.DMA((2,)),
            pltpu.SemaphoreType.DMA((2,)),
            pltpu.SemaphoreType.DMA,
        ],
        compiler_params=pltpu.CompilerParams(
            collective_id=0, vmem_limit_bytes=63 * 1024 * 1024,
        ),
    )(pb, resid, gamma)
